# baseline (speedup 1.0000x reference)
import jax
import jax.numpy as jnp
from jax import lax
from jax.experimental import pallas as pl
from jax.experimental.pallas import tpu as pltpu
from jax.experimental.pallas import tpu_sc as plsc

_MAX_DEGREE = 512
_HIDDEN = 128
_N = 100000

_NC = 2
_NS = 16
_NW = _NC * _NS

_CHUNK = 112
_FULL = 3136
_LAST = 2784
_TAIL_BASE = 31 * _FULL + _LAST - 96
_TAIL = _N - _TAIL_BASE


def _body(deg_hbm, table_hbm, out_hbm,
          bounce, table_sh, table_v, idxa, wb0, wb1, gs0, ws0, ws1, ts):
    c = lax.axis_index("c")
    s = lax.axis_index("s")
    wid = s * _NC + c
    base = wid * _FULL
    last = wid == _NW - 1

    rows_per = _MAX_DEGREE // _NS
    pltpu.sync_copy(table_hbm.at[pl.ds(s * rows_per, rows_per)], bounce)
    pltpu.sync_copy(bounce, table_sh.at[pl.ds(s * rows_per, rows_per)])

    @pl.when(jnp.logical_not(last))
    def _():
        pltpu.sync_copy(deg_hbm.at[pl.ds(base, _FULL)], idxa)

    @pl.when(last)
    def _():
        pltpu.sync_copy(deg_hbm.at[pl.ds(base, _LAST)], idxa.at[pl.ds(0, _LAST)])
        pltpu.sync_copy(deg_hbm.at[pl.ds(_TAIL_BASE, _TAIL)],
                        idxa.at[pl.ds(_LAST, _TAIL)])

    def clamp(lo, hi):
        for i in range(lo, hi):
            sl = pl.ds(i * 16, 16)
            idxa[sl] = jnp.minimum(jnp.maximum(idxa[sl], 0), _MAX_DEGREE - 1)

    clamp(0, (_LAST + _TAIL) // 16)

    @pl.when(jnp.logical_not(last))
    def _():
        clamp((_LAST + _TAIL) // 16, _FULL // 16)

    plsc.subcore_barrier()
    pltpu.sync_copy(table_sh, table_v)

    def fire_gather(idx_off, wb, sem):
        return pltpu.async_copy(
            table_sh.at[idxa.at[pl.ds(idx_off, _CHUNK)]], wb, sem)

    def compute_gather(idx_off, wb, nrows):
        @plsc.parallel_loop(0, nrows // 16)
        def _(rg):
            vv = idxa[pl.ds(idx_off + rg * 16, 16)]
            for lane in range(16):
                tb = vv[lane]
                ob = rg * 16 + lane
                for c8 in range(_HIDDEN // 16):
                    wb[ob, pl.ds(c8 * 16, 16)] = (
                        table_v[tb, pl.ds(c8 * 16, 16)])

    def fire_write(row_off, wb, sem):
        return pltpu.async_copy(
            wb, out_hbm.at[pl.ds(row_off, _CHUNK)], sem)

    def drain_write(sem):
        pltpu.make_async_copy(
            wb0, out_hbm.at[pl.ds(0, _CHUNK)], sem).wait()

    g0 = fire_gather(0, wb0, gs0)
    compute_gather(_CHUNK, wb1, _CHUNK)
    fire_write(base + _CHUNK, wb1, ws1)
    g0.wait()
    fire_write(base, wb0, ws0)

    n_pairs = jnp.where(last, 12, 14)

    @pl.loop(1, n_pairs)
    def _(t):
        off = t * 2 * _CHUNK
        drain_write(ws0)
        ga = fire_gather(off, wb0, gs0)
        drain_write(ws1)
        compute_gather(off + _CHUNK, wb1, _CHUNK)
        fire_write(base + off + _CHUNK, wb1, ws1)
        ga.wait()
        fire_write(base + off, wb0, ws0)

    @pl.when(jnp.logical_not(last))
    def _():
        drain_write(ws0)
        drain_write(ws1)

    @pl.when(last)
    def _():
        drain_write(ws0)
        compute_gather(_LAST, wb0, _TAIL)
        pltpu.async_copy(
            wb0.at[pl.ds(0, _TAIL)],
            out_hbm.at[pl.ds(_TAIL_BASE, _TAIL)], ts).wait()
        drain_write(ws1)


@jax.jit
def _run(degrees, table):
    mesh = plsc.VectorSubcoreMesh(core_axis_name="c", subcore_axis_name="s")
    k = pl.kernel(
        _body,
        mesh=mesh,
        compiler_params=pltpu.CompilerParams(needs_layout_passes=False),
        out_type=jax.ShapeDtypeStruct((_N, _HIDDEN), jnp.float32),
        scratch_types=[
            pltpu.VMEM((_MAX_DEGREE // _NS, _HIDDEN), jnp.float32),
            pltpu.VMEM_SHARED((_MAX_DEGREE, _HIDDEN), jnp.float32),
            pltpu.VMEM((_MAX_DEGREE, _HIDDEN), jnp.float32),
            pltpu.VMEM((_FULL,), jnp.int32),
            pltpu.VMEM((_CHUNK, _HIDDEN), jnp.float32),
            pltpu.VMEM((_CHUNK, _HIDDEN), jnp.float32),
            pltpu.SemaphoreType.DMA,
            pltpu.SemaphoreType.DMA,
            pltpu.SemaphoreType.DMA,
            pltpu.SemaphoreType.DMA,
        ],
    )
    return k(degrees, table)


def kernel(degrees, degree_embedding):
    return _run(degrees.astype(jnp.int32), degree_embedding)


# speedup vs baseline: 1.4310x; 1.4310x over previous
import jax
import jax.numpy as jnp
from jax import lax
from jax.experimental import pallas as pl
from jax.experimental.pallas import tpu as pltpu
from jax.experimental.pallas import tpu_sc as plsc

_MAX_DEGREE = 512
_HIDDEN = 128
_N = 100000

_NC = 2
_NS = 16
_NW = _NC * _NS

_CHUNK = 112
_NB = 4
_FULL = 3136
_LAST = 2784
_TAIL_BASE = 31 * _FULL + _LAST - 96
_TAIL = _N - _TAIL_BASE


def _body(deg_hbm, table_hbm, out_hbm,
          bounce, table_sh, idxa, wb0, wb1, wb2, wb3,
          sg, gs0, gs1, gs2, gs3, ws0, ws1, ws2, ws3, ts):
    c = lax.axis_index("c")
    s = lax.axis_index("s")
    wid = s * _NC + c
    base = wid * _FULL
    last = wid == _NW - 1

    wb = [wb0, wb1, wb2, wb3]
    gs = [gs0, gs1, gs2, gs3]
    ws = [ws0, ws1, ws2, ws3]

    rows_per = _MAX_DEGREE // _NS
    hstage = pltpu.async_copy(
        table_hbm.at[pl.ds(s * rows_per, rows_per)], bounce, sg)

    @pl.when(jnp.logical_not(last))
    def _():
        pltpu.sync_copy(deg_hbm.at[pl.ds(base, _FULL)], idxa)

    @pl.when(last)
    def _():
        pltpu.sync_copy(deg_hbm.at[pl.ds(base, _LAST)], idxa.at[pl.ds(0, _LAST)])
        pltpu.sync_copy(deg_hbm.at[pl.ds(_TAIL_BASE, _TAIL)],
                        idxa.at[pl.ds(_LAST, _TAIL)])

    hstage.wait()
    pltpu.sync_copy(bounce, table_sh.at[pl.ds(s * rows_per, rows_per)])

    def clamp(lo, hi):
        for i in range(lo, hi):
            sl = pl.ds(i * 16, 16)
            idxa[sl] = jnp.minimum(jnp.maximum(idxa[sl], 0), _MAX_DEGREE - 1)

    clamp(0, (_LAST + _TAIL) // 16)

    @pl.when(jnp.logical_not(last))
    def _():
        clamp((_LAST + _TAIL) // 16, _FULL // 16)

    plsc.subcore_barrier()

    def fire_gather(idx_off, b):
        return pltpu.async_copy(
            table_sh.at[idxa.at[pl.ds(idx_off, _CHUNK)]], wb[b], gs[b])

    def fire_write(row_off, b):
        return pltpu.async_copy(
            wb[b], out_hbm.at[pl.ds(row_off, _CHUNK)], ws[b])

    def drain_write(b):
        pltpu.make_async_copy(
            wb0, out_hbm.at[pl.ds(0, _CHUNK)], ws[b]).wait()

    gh = [fire_gather(b * _CHUNK, b) for b in range(_NB)]
    for b in range(_NB):
        gh[b].wait()
        fire_write(base + b * _CHUNK, b)

    n_quads = jnp.where(last, 6, 7)

    @pl.loop(1, n_quads)
    def _(t):
        off = t * _NB * _CHUNK
        hs = []
        for b in range(_NB):
            drain_write(b)
            hs.append(fire_gather(off + b * _CHUNK, b))
        for b in range(_NB):
            hs[b].wait()
            fire_write(base + off + b * _CHUNK, b)

    @pl.when(jnp.logical_not(last))
    def _():
        for b in range(_NB):
            drain_write(b)

    @pl.when(last)
    def _():
        drain_write(0)
        pltpu.async_copy(
            table_sh.at[idxa.at[pl.ds(_LAST, _TAIL)]],
            wb0.at[pl.ds(0, _TAIL)], gs0).wait()
        pltpu.async_copy(
            wb0.at[pl.ds(0, _TAIL)],
            out_hbm.at[pl.ds(_TAIL_BASE, _TAIL)], ts).wait()
        for b in range(1, _NB):
            drain_write(b)


@jax.jit
def _run(degrees, table):
    mesh = plsc.VectorSubcoreMesh(core_axis_name="c", subcore_axis_name="s")
    k = pl.kernel(
        _body,
        mesh=mesh,
        compiler_params=pltpu.CompilerParams(needs_layout_passes=False),
        out_type=jax.ShapeDtypeStruct((_N, _HIDDEN), jnp.float32),
        scratch_types=[
            pltpu.VMEM((_MAX_DEGREE // _NS, _HIDDEN), jnp.float32),
            pltpu.VMEM_SHARED((_MAX_DEGREE, _HIDDEN), jnp.float32),
            pltpu.VMEM((_FULL,), jnp.int32),
            pltpu.VMEM((_CHUNK, _HIDDEN), jnp.float32),
            pltpu.VMEM((_CHUNK, _HIDDEN), jnp.float32),
            pltpu.VMEM((_CHUNK, _HIDDEN), jnp.float32),
            pltpu.VMEM((_CHUNK, _HIDDEN), jnp.float32),
            pltpu.SemaphoreType.DMA,
            pltpu.SemaphoreType.DMA,
            pltpu.SemaphoreType.DMA,
            pltpu.SemaphoreType.DMA,
            pltpu.SemaphoreType.DMA,
            pltpu.SemaphoreType.DMA,
            pltpu.SemaphoreType.DMA,
            pltpu.SemaphoreType.DMA,
            pltpu.SemaphoreType.DMA,
            pltpu.SemaphoreType.DMA,
        ],
    )
    return k(degrees, table)


def kernel(degrees, degree_embedding):
    return _run(degrees.astype(jnp.int32), degree_embedding)
